# baseline (speedup 1.0000x reference)
import jax
import jax.numpy as jnp
from jax import lax
from jax.experimental import pallas as pl
from jax.experimental.pallas import tpu as pltpu
from jax.experimental.pallas import tpu_sc as plsc

_LANES = 16
_NC = 2
_NS = 16
_NW = _NC * _NS


def _weff_sparsecore(W_in, dendrites, WdT, U, C, SEQL):
    rows_per_w = U // _NW

    mesh = plsc.VectorSubcoreMesh(core_axis_name="c", subcore_axis_name="s")

    def body(win_hbm, d_hbm, wdt_hbm, weff_hbm, wdt_v, win_v, d_v, eff_v,
             sem_wdt, sem_win, sem_d):
        wid = lax.axis_index("s") * _NC + lax.axis_index("c")
        base_u = wid * rows_per_w
        cp_wdt = pltpu.async_copy(
            wdt_hbm.at[pl.ds(base_u, rows_per_w), :], wdt_v, sem_wdt)
        cp_win = pltpu.async_copy(
            win_hbm.at[pl.ds(base_u, rows_per_w), :], win_v, sem_win)
        cp_d = pltpu.async_copy(
            d_hbm.at[pl.ds(base_u, rows_per_w), :], d_v, sem_d)
        cp_wdt.wait()
        cp_win.wait()
        cp_d.wait()
        @plsc.parallel_loop(0, rows_per_w * C, step=_LANES, unroll=4)
        def _(i):
            r = i // C
            col = i - r * C
            r_vec = jnp.full((_LANES,), r, jnp.int32)
            d = d_v[r, pl.ds(col, _LANES)]
            g = plsc.load_gather(wdt_v, [r_vec, d])
            w = win_v[r, pl.ds(col, _LANES)]
            eff_v[r, pl.ds(col, _LANES)] = jnp.maximum(w, 5e-5) * g

        pltpu.sync_copy(eff_v, weff_hbm.at[pl.ds(base_u, rows_per_w), :])

    return pl.kernel(
        body,
        out_type=jax.ShapeDtypeStruct((U, C), jnp.float32),
        mesh=mesh,
        compiler_params=pltpu.CompilerParams(needs_layout_passes=False),
        scratch_types=[
            pltpu.VMEM((rows_per_w, SEQL), jnp.float32),
            pltpu.VMEM((rows_per_w, C), jnp.float32),
            pltpu.VMEM((rows_per_w, C), jnp.int32),
            pltpu.VMEM((rows_per_w, C), jnp.float32),
            pltpu.SemaphoreType.DMA,
            pltpu.SemaphoreType.DMA,
            pltpu.SemaphoreType.DMA,
        ],
    )(W_in, dendrites, WdT)


def _mm_body(x_ref, w_ref, b_ref, o_ref):
    acc = lax.dot_general(
        x_ref[...], w_ref[...], (((1,), (1,)), ((), ())),
        preferred_element_type=jnp.float32,
    )
    o_ref[...] = jnp.maximum(acc + b_ref[...], 0.0)


def kernel(x, W_in, Wd, b, dendrites):
    B, C = x.shape
    U = W_in.shape[0]
    SEQL = Wd.shape[0]

    weff = _weff_sparsecore(W_in, dendrites, Wd.T, U, C, SEQL)

    blk = B // 2
    return pl.pallas_call(
        _mm_body,
        grid=(2,),
        in_specs=[
            pl.BlockSpec((blk, C), lambda i: (i, 0)),
            pl.BlockSpec((U, C), lambda i: (0, 0)),
            pl.BlockSpec((1, U), lambda i: (0, 0)),
        ],
        out_specs=pl.BlockSpec((blk, U), lambda i: (i, 0)),
        out_shape=jax.ShapeDtypeStruct((B, U), jnp.float32),
    )(x, weff, b.reshape(1, U))


# speedup vs baseline: 1.0216x; 1.0216x over previous
import jax
import jax.numpy as jnp
from jax import lax
from jax.experimental import pallas as pl
from jax.experimental.pallas import tpu as pltpu
from jax.experimental.pallas import tpu_sc as plsc

_LANES = 16
_NC = 2
_NS = 16
_NW = _NC * _NS


def _gather_sparsecore(dendrites, WdT, U, C, SEQL):
    rows_per_w = U // _NW

    mesh = plsc.VectorSubcoreMesh(core_axis_name="c", subcore_axis_name="s")

    def body(d_hbm, wdt_hbm, g_hbm, wdt_v, d_v, g_v, sem_wdt, sem_d):
        wid = lax.axis_index("s") * _NC + lax.axis_index("c")
        base_u = wid * rows_per_w
        cp_wdt = pltpu.async_copy(
            wdt_hbm.at[pl.ds(base_u, rows_per_w), :], wdt_v, sem_wdt)
        cp_d = pltpu.async_copy(
            d_hbm.at[pl.ds(base_u, rows_per_w), :], d_v, sem_d)
        cp_wdt.wait()
        cp_d.wait()

        @plsc.parallel_loop(0, rows_per_w * C, step=_LANES, unroll=4)
        def _(i):
            r = i // C
            col = i - r * C
            r_vec = jnp.full((_LANES,), r, jnp.int32)
            d = d_v[r, pl.ds(col, _LANES)]
            g_v[r, pl.ds(col, _LANES)] = plsc.load_gather(wdt_v, [r_vec, d])

        pltpu.sync_copy(g_v, g_hbm.at[pl.ds(base_u, rows_per_w), :])

    return pl.kernel(
        body,
        out_type=jax.ShapeDtypeStruct((U, C), jnp.float32),
        mesh=mesh,
        compiler_params=pltpu.CompilerParams(needs_layout_passes=False),
        scratch_types=[
            pltpu.VMEM((rows_per_w, SEQL), jnp.float32),
            pltpu.VMEM((rows_per_w, C), jnp.int32),
            pltpu.VMEM((rows_per_w, C), jnp.float32),
            pltpu.SemaphoreType.DMA,
            pltpu.SemaphoreType.DMA,
        ],
    )(dendrites, WdT)


def _mm_body(x_ref, win_ref, g_ref, b_ref, o_ref):
    weff = jnp.maximum(win_ref[...], 5e-5) * g_ref[...]
    acc = lax.dot_general(
        x_ref[...], weff, (((1,), (1,)), ((), ())),
        preferred_element_type=jnp.float32,
    )
    o_ref[...] = jnp.maximum(acc + b_ref[...], 0.0)


def kernel(x, W_in, Wd, b, dendrites):
    B, C = x.shape
    U = W_in.shape[0]
    SEQL = Wd.shape[0]

    g = _gather_sparsecore(dendrites, Wd.T, U, C, SEQL)

    return pl.pallas_call(
        _mm_body,
        out_shape=jax.ShapeDtypeStruct((B, U), jnp.float32),
    )(x, W_in, g, b.reshape(1, U))
